# Initial kernel scaffold; baseline (speedup 1.0000x reference)
import jax
import jax.numpy as jnp
from jax.experimental import pallas as pl


def kernel(token_ids, weight):
    raise NotImplementedError("write your pallas kernel here")


# speedup vs baseline: 1.6848x; 1.6848x over previous
import functools

import jax
import jax.numpy as jnp
from jax import lax
from jax.experimental import pallas as pl
from jax.experimental.pallas import tpu as pltpu
from jax.experimental.pallas import tpu_sc as plsc

_INFO = plsc.get_sparse_core_info()
_NC = _INFO.num_cores
_NS = _INFO.num_subcores
_NW = _NC * _NS

_B = 16384 * 50
_D = 64
_ROW = 128
_NROWS = _B // _ROW
_RPW = _NROWS // _NW


def _body(tok_hbm, w_hbm, out_hbm, idx_v, rows_v, gsem):
    wid = lax.axis_index("s") * _NC + lax.axis_index("c")
    row0 = wid * _RPW
    pltpu.sync_copy(tok_hbm.at[pl.ds(row0, _RPW)], idx_v)

    def step(j, carry):
        pltpu.async_copy(w_hbm.at[idx_v.at[j]], rows_v, gsem).wait()
        pltpu.sync_copy(rows_v, out_hbm.at[pl.ds((row0 + j) * _ROW, _ROW)])
        return carry

    lax.fori_loop(0, _RPW, step, 0)


def kernel(token_ids, weight):
    tok = token_ids.reshape(_NROWS, _ROW)
    mesh = plsc.VectorSubcoreMesh(core_axis_name="c", subcore_axis_name="s")
    out = pl.kernel(
        _body,
        mesh=mesh,
        compiler_params=pltpu.CompilerParams(use_tc_tiling_on_sc=False),
        out_type=jax.ShapeDtypeStruct((_B, _D), jnp.float32),
        scratch_types=[
            pltpu.VMEM((_RPW, _ROW), jnp.int32),
            pltpu.VMEM((_ROW, _D), jnp.float32),
            pltpu.SemaphoreType.DMA,
        ],
    )(tok, weight)
    return out.reshape(token_ids.shape + (_D,))


# speedup vs baseline: 1.8587x; 1.1032x over previous
import functools

import jax
import jax.numpy as jnp
from jax import lax
from jax.experimental import pallas as pl
from jax.experimental.pallas import tpu as pltpu
from jax.experimental.pallas import tpu_sc as plsc

_INFO = plsc.get_sparse_core_info()
_NC = _INFO.num_cores
_NS = _INFO.num_subcores
_NW = _NC * _NS

_B = 16384 * 50
_D = 64
_ROW = 1024
_NROWS = _B // _ROW
_RPW = _NROWS // _NW


def _body(tok_hbm, w_hbm, out_hbm, idx_v, rows_v, gsem):
    wid = lax.axis_index("s") * _NC + lax.axis_index("c")
    row0 = wid * _RPW
    pltpu.sync_copy(tok_hbm.at[pl.ds(row0, _RPW)], idx_v)

    def step(j, carry):
        pltpu.async_copy(w_hbm.at[idx_v.at[j]], rows_v, gsem).wait()
        pltpu.sync_copy(rows_v, out_hbm.at[pl.ds((row0 + j) * _ROW, _ROW)])
        return carry

    lax.fori_loop(0, _RPW, step, 0)


def kernel(token_ids, weight):
    tok = token_ids.reshape(_NROWS, _ROW)
    mesh = plsc.VectorSubcoreMesh(core_axis_name="c", subcore_axis_name="s")
    out = pl.kernel(
        _body,
        mesh=mesh,
        compiler_params=pltpu.CompilerParams(use_tc_tiling_on_sc=False),
        out_type=jax.ShapeDtypeStruct((_B, _D), jnp.float32),
        scratch_types=[
            pltpu.VMEM((_RPW, _ROW), jnp.int32),
            pltpu.VMEM((_ROW, _D), jnp.float32),
            pltpu.SemaphoreType.DMA,
        ],
    )(tok, weight)
    return out.reshape(token_ids.shape + (_D,))


# speedup vs baseline: 1.8750x; 1.0088x over previous
import jax
import jax.numpy as jnp
from jax import lax
from jax.experimental import pallas as pl
from jax.experimental.pallas import tpu as pltpu
from jax.experimental.pallas import tpu_sc as plsc

_INFO = plsc.get_sparse_core_info()
_NC = _INFO.num_cores
_NS = _INFO.num_subcores
_NW = _NC * _NS

_B = 16384 * 50
_D = 64
_G = 256
_NBUF = 4
_LOOK = _NBUF - 1
_NG = _B // _G // _NW
_NGROUPS = _B // _G


def _body(tok_hbm, w_hbm, out_hbm, idx_v, b0, b1, b2, b3,
          g0s, g1s, g2s, g3s, w0s, w1s, w2s, w3s):
    bufs = (b0, b1, b2, b3)
    gsems = (g0s, g1s, g2s, g3s)
    wsems = (w0s, w1s, w2s, w3s)

    wid = lax.axis_index("s") * _NC + lax.axis_index("c")
    grp0 = wid * _NG
    pltpu.sync_copy(tok_hbm.at[pl.ds(grp0, _NG)], idx_v)

    def fire(g, b):
        pltpu.async_copy(w_hbm.at[idx_v.at[g]], bufs[b], gsems[b])

    for g in range(_LOOK):
        fire(g, g)

    def outer(t, carry):
        for b in range(_NBUF):
            s = t * _NBUF + b
            nb = (b + _LOOK) % _NBUF

            @pl.when(s + _LOOK < _NG)
            def _():
                @pl.when(s + _LOOK >= _NBUF)
                def _():
                    pltpu.make_async_copy(
                        bufs[nb], out_hbm.at[pl.ds(0, _G)], wsems[nb]).wait()
                fire(s + _LOOK, nb)

            pltpu.make_async_copy(
                w_hbm.at[idx_v.at[s]], bufs[b], gsems[b]).wait()
            pltpu.async_copy(
                bufs[b], out_hbm.at[pl.ds((grp0 + s) * _G, _G)], wsems[b])
        return carry

    lax.fori_loop(0, _NG // _NBUF, outer, 0)

    for b in range(_NBUF):
        pltpu.make_async_copy(
            bufs[b], out_hbm.at[pl.ds(0, _G)], wsems[b]).wait()


def kernel(token_ids, weight):
    tok = token_ids.reshape(_NGROUPS, _G)
    mesh = plsc.VectorSubcoreMesh(core_axis_name="c", subcore_axis_name="s")
    out = pl.kernel(
        _body,
        mesh=mesh,
        compiler_params=pltpu.CompilerParams(use_tc_tiling_on_sc=False),
        out_type=jax.ShapeDtypeStruct((_B, _D), jnp.float32),
        scratch_types=[
            pltpu.VMEM((_NG, _G), jnp.int32)]
        + [pltpu.VMEM((_G, _D), jnp.float32) for _ in range(_NBUF)]
        + [pltpu.SemaphoreType.DMA for _ in range(2 * _NBUF)],
    )(tok, weight)
    return out.reshape(token_ids.shape + (_D,))
